# baseline (speedup 1.0000x reference)
import functools

import jax
import jax.numpy as jnp
from jax import lax
from jax.experimental import pallas as pl
from jax.experimental.pallas import tpu as pltpu
from jax.experimental.pallas import tpu_sc as plsc

BATCH = 16384
EMB = 64
N_BRAND = 1000
_NC, _NS = 2, 16
_NW = _NC * _NS
_BPW = BATCH // _NW
_NS1 = 288


def _gather2(user_ids, device_ids, user_table, device_table):
  mesh = plsc.VectorSubcoreMesh(core_axis_name="c", subcore_axis_name="s")
  out_t = [jax.ShapeDtypeStruct((BATCH, EMB), jnp.float32) for _ in range(2)]

  @functools.partial(
      pl.kernel,
      out_type=out_t,
      mesh=mesh,
      scratch_types=[
          pltpu.VMEM((_BPW,), jnp.int32),
          pltpu.VMEM((_BPW,), jnp.int32),
          pltpu.VMEM((_NS1, EMB), jnp.float32),
          pltpu.VMEM((_NS1, EMB), jnp.float32),
          pltpu.SemaphoreType.DMA,
          pltpu.SemaphoreType.DMA,
      ],
  )
  def k(uid_hbm, did_hbm, ut_hbm, dt_hbm, ou_hbm, od_hbm,
        uidx, didx, stag_u, stag_d, sem_s, sem_d):
    wid = lax.axis_index("s") * _NC + lax.axis_index("c")
    base = wid * _BPW
    pltpu.sync_copy(uid_hbm.at[pl.ds(base, _BPW)], uidx)
    pltpu.sync_copy(did_hbm.at[pl.ds(base, _BPW)], didx)

    for idx_v, tab, out, stag in ((uidx, ut_hbm, ou_hbm, stag_u),
                                  (didx, dt_hbm, od_hbm, stag_d)):
      def fire_stream(g, carry):
        vec = idx_v[pl.ds(g * 16, 16)]
        for l in range(16):
          pltpu.async_copy(tab.at[vec[l]], stag.at[g * 16 + l], sem_s)
        return carry
      lax.fori_loop(0, _NS1 // 16, fire_stream, 0)

      def fire_dma(g, carry):
        vec = idx_v[pl.ds(_NS1 + g * 16, 16)]
        for l in range(16):
          i = _NS1 + g * 16 + l
          pltpu.async_copy(tab.at[vec[l]], out.at[base + i], sem_d)
        return carry
      lax.fori_loop(0, (_BPW - _NS1) // 16, fire_dma, 0)

    pltpu.make_async_copy(ut_hbm.at[pl.ds(0, _NS1)], stag_u, sem_s).wait()
    pltpu.make_async_copy(dt_hbm.at[pl.ds(0, _NS1)], stag_d, sem_s).wait()
    for tab, out in ((ut_hbm, ou_hbm), (dt_hbm, od_hbm)):
      pltpu.make_async_copy(tab.at[pl.ds(0, _BPW - _NS1)],
                            out.at[pl.ds(0, _BPW - _NS1)], sem_d).wait()
    pltpu.sync_copy(stag_u, ou_hbm.at[pl.ds(base, _NS1)])
    pltpu.sync_copy(stag_d, od_hbm.at[pl.ds(base, _NS1)])

  return k(user_ids, device_ids, user_table, device_table)


_TB = 2048


def _mlp_body(u_ref, d_ref, bid_ref, bt_ref, w1u_ref, w1d_ref, w1b_ref,
              b1_ref, w2_ref, b2_ref, w3_ref, b3_ref, o_ref):
  iota = lax.broadcasted_iota(jnp.int32, (_TB, 1024), 1)
  onehot = (bid_ref[...].reshape(_TB, 1) == iota).astype(jnp.float32)
  b = jnp.dot(onehot, bt_ref[...], preferred_element_type=jnp.float32)
  h = jnp.dot(u_ref[...], w1u_ref[...], preferred_element_type=jnp.float32)
  h = h + jnp.dot(d_ref[...], w1d_ref[...], preferred_element_type=jnp.float32)
  h = h + jnp.dot(b, w1b_ref[...], preferred_element_type=jnp.float32)
  h = jnp.maximum(h + b1_ref[...], 0.0)
  h2 = jnp.dot(h, w2_ref[...], preferred_element_type=jnp.float32)
  h2 = jnp.maximum(h2 + b2_ref[...], 0.0)
  o_ref[...] = jnp.sum(h2 * w3_ref[...], axis=1) + b3_ref[0, 0]


def _mlp(u, d, brand_ids, brand_table, W1, b1, W2, b2, W3, b3):
  w1u, w1d, w1b = W1[:EMB], W1[EMB:2 * EMB], W1[2 * EMB:]
  bt_pad = jnp.zeros((1024, EMB), jnp.float32).at[:N_BRAND].set(brand_table)
  grid = (BATCH // _TB,)
  full = lambda shape: pl.BlockSpec(shape, lambda i: (0, 0))
  tile = pl.BlockSpec((_TB, EMB), lambda i: (i, 0))
  return pl.pallas_call(
      _mlp_body,
      grid=grid,
      in_specs=[
          tile, tile,
          pl.BlockSpec((_TB,), lambda i: (i,)),
          full((1024, EMB)),
          full((EMB, 128)), full((EMB, 128)), full((EMB, 128)),
          full((1, 128)),
          full((128, 64)), full((1, 64)),
          full((1, 64)), full((1, 1)),
      ],
      out_specs=pl.BlockSpec((_TB,), lambda i: (i,)),
      out_shape=jax.ShapeDtypeStruct((BATCH,), jnp.float32),
  )(u, d, brand_ids, bt_pad, w1u, w1d, w1b, b1.reshape(1, 128), W2,
    b2.reshape(1, 64), W3.reshape(1, EMB), b3.reshape(1, 1))


def kernel(user_ids, device_ids, brand_ids, user_table, device_table,
           brand_table, W1, b1, W2, b2, W3, b3):
  u, d = _gather2(user_ids.astype(jnp.int32), device_ids.astype(jnp.int32),
                  user_table, device_table)
  return _mlp(u, d, brand_ids.astype(jnp.int32), brand_table,
              W1, b1, W2, b2, W3, b3)


# speedup vs baseline: 1.1563x; 1.1563x over previous
import functools

import jax
import jax.numpy as jnp
from jax import lax
from jax.experimental import pallas as pl
from jax.experimental.pallas import tpu as pltpu
from jax.experimental.pallas import tpu_sc as plsc

BATCH = 16384
EMB = 64
N_BRAND = 1000
_NC, _NS = 2, 16
_NW = _NC * _NS
_BPW = BATCH // _NW
_NS1 = 384


def _gather2(user_ids, device_ids, user_table, device_table):
  mesh = plsc.VectorSubcoreMesh(core_axis_name="c", subcore_axis_name="s")
  out_t = [jax.ShapeDtypeStruct((BATCH, EMB), jnp.float32) for _ in range(2)]

  @functools.partial(
      pl.kernel,
      out_type=out_t,
      mesh=mesh,
      scratch_types=[
          pltpu.VMEM((_BPW,), jnp.int32),
          pltpu.VMEM((_BPW,), jnp.int32),
          pltpu.VMEM((_NS1, EMB), jnp.float32),
          pltpu.VMEM((_NS1, EMB), jnp.float32),
          pltpu.SemaphoreType.DMA,
          pltpu.SemaphoreType.DMA,
      ],
  )
  def k(uid_hbm, did_hbm, ut_hbm, dt_hbm, ou_hbm, od_hbm,
        uidx, didx, stag_u, stag_d, sem_s, sem_d):
    wid = lax.axis_index("s") * _NC + lax.axis_index("c")
    base = wid * _BPW
    pltpu.sync_copy(uid_hbm.at[pl.ds(base, _BPW)], uidx)
    pltpu.sync_copy(did_hbm.at[pl.ds(base, _BPW)], didx)

    for idx_v, tab, out, stag in ((uidx, ut_hbm, ou_hbm, stag_u),
                                  (didx, dt_hbm, od_hbm, stag_d)):
      def fire_stream(g, carry):
        vec = idx_v[pl.ds(g * 16, 16)]
        for l in range(16):
          pltpu.async_copy(tab.at[vec[l]], stag.at[g * 16 + l], sem_s)
        return carry
      lax.fori_loop(0, _NS1 // 16, fire_stream, 0)

    for idx_v, tab, out in ((uidx, ut_hbm, ou_hbm), (didx, dt_hbm, od_hbm)):
      def fire_dma(g, carry):
        vec = idx_v[pl.ds(_NS1 + g * 16, 16)]
        for l in range(16):
          i = _NS1 + g * 16 + l
          pltpu.async_copy(tab.at[vec[l]], out.at[base + i], sem_d)
        return carry
      lax.fori_loop(0, (_BPW - _NS1) // 16, fire_dma, 0)

    pltpu.make_async_copy(ut_hbm.at[pl.ds(0, _NS1)], stag_u, sem_s).wait()
    pltpu.make_async_copy(dt_hbm.at[pl.ds(0, _NS1)], stag_d, sem_s).wait()
    for tab, out in ((ut_hbm, ou_hbm), (dt_hbm, od_hbm)):
      pltpu.make_async_copy(tab.at[pl.ds(0, _BPW - _NS1)],
                            out.at[pl.ds(0, _BPW - _NS1)], sem_d).wait()
    pltpu.sync_copy(stag_u, ou_hbm.at[pl.ds(base, _NS1)])
    pltpu.sync_copy(stag_d, od_hbm.at[pl.ds(base, _NS1)])

  return k(user_ids, device_ids, user_table, device_table)


_TB = 2048


def _mlp_body(u_ref, d_ref, bid_ref, bt_ref, w1u_ref, w1d_ref, w1b_ref,
              b1_ref, w2_ref, b2_ref, w3_ref, b3_ref, o_ref):
  iota = lax.broadcasted_iota(jnp.int32, (_TB, 1024), 1)
  onehot = (bid_ref[...].reshape(_TB, 1) == iota).astype(jnp.float32)
  b = jnp.dot(onehot, bt_ref[...], preferred_element_type=jnp.float32)
  h = jnp.dot(u_ref[...], w1u_ref[...], preferred_element_type=jnp.float32)
  h = h + jnp.dot(d_ref[...], w1d_ref[...], preferred_element_type=jnp.float32)
  h = h + jnp.dot(b, w1b_ref[...], preferred_element_type=jnp.float32)
  h = jnp.maximum(h + b1_ref[...], 0.0)
  h2 = jnp.dot(h, w2_ref[...], preferred_element_type=jnp.float32)
  h2 = jnp.maximum(h2 + b2_ref[...], 0.0)
  o_ref[...] = jnp.sum(h2 * w3_ref[...], axis=1) + b3_ref[0, 0]


def _mlp(u, d, brand_ids, brand_table, W1, b1, W2, b2, W3, b3):
  w1u, w1d, w1b = W1[:EMB], W1[EMB:2 * EMB], W1[2 * EMB:]
  bt_pad = jnp.zeros((1024, EMB), jnp.float32).at[:N_BRAND].set(brand_table)
  grid = (BATCH // _TB,)
  full = lambda shape: pl.BlockSpec(shape, lambda i: (0, 0))
  tile = pl.BlockSpec((_TB, EMB), lambda i: (i, 0))
  return pl.pallas_call(
      _mlp_body,
      grid=grid,
      in_specs=[
          tile, tile,
          pl.BlockSpec((_TB,), lambda i: (i,)),
          full((1024, EMB)),
          full((EMB, 128)), full((EMB, 128)), full((EMB, 128)),
          full((1, 128)),
          full((128, 64)), full((1, 64)),
          full((1, 64)), full((1, 1)),
      ],
      out_specs=pl.BlockSpec((_TB,), lambda i: (i,)),
      out_shape=jax.ShapeDtypeStruct((BATCH,), jnp.float32),
  )(u, d, brand_ids, bt_pad, w1u, w1d, w1b, b1.reshape(1, 128), W2,
    b2.reshape(1, 64), W3.reshape(1, EMB), b3.reshape(1, 1))


def kernel(user_ids, device_ids, brand_ids, user_table, device_table,
           brand_table, W1, b1, W2, b2, W3, b3):
  u, d = _gather2(user_ids.astype(jnp.int32), device_ids.astype(jnp.int32),
                  user_table, device_table)
  return _mlp(u, d, brand_ids.astype(jnp.int32), brand_table,
              W1, b1, W2, b2, W3, b3)


# speedup vs baseline: 1.4680x; 1.2696x over previous
import functools

import jax
import jax.numpy as jnp
from jax import lax
from jax.experimental import pallas as pl
from jax.experimental.pallas import tpu as pltpu
from jax.experimental.pallas import tpu_sc as plsc

BATCH = 16384
EMB = 64
N_BRAND = 1000
_NC, _NS = 2, 16
_NW = _NC * _NS
_BPW = BATCH // _NW
_NS1 = 384


def _gather2(user_ids, device_ids, user_table, device_table):
  mesh = plsc.VectorSubcoreMesh(core_axis_name="c", subcore_axis_name="s")
  out_t = [jax.ShapeDtypeStruct((BATCH, EMB), jnp.float32) for _ in range(2)]

  @functools.partial(
      pl.kernel,
      out_type=out_t,
      mesh=mesh,
      scratch_types=[
          pltpu.VMEM((_BPW,), jnp.int32),
          pltpu.VMEM((_BPW,), jnp.int32),
          pltpu.VMEM((_BPW, EMB), jnp.float32),
          pltpu.SemaphoreType.DMA,
          pltpu.SemaphoreType.DMA,
          pltpu.SemaphoreType.DMA,
          pltpu.SemaphoreType.DMA,
      ],
  )
  def k(uid_hbm, did_hbm, ut_hbm, dt_hbm, ou_hbm, od_hbm,
        uidx, didx, stag, s0, s1, s2, s3):
    sems = (s0, s1, s2, s3)
    wid = lax.axis_index("s") * _NC + lax.axis_index("c")
    base = wid * _BPW
    pltpu.sync_copy(uid_hbm.at[pl.ds(base, _BPW)], uidx)
    pltpu.sync_copy(did_hbm.at[pl.ds(base, _BPW)], didx)

    for idx_v, tab, out in ((uidx, ut_hbm, ou_hbm), (didx, dt_hbm, od_hbm)):
      def fire(g, carry):
        vec = idx_v[pl.ds(g * 16, 16)]
        for l in range(16):
          pltpu.async_copy(tab.at[vec[l]], stag.at[g * 16 + l], sems[l % 4])
        return carry
      lax.fori_loop(0, _BPW // 16, fire, 0)

      for q in range(4):
        pltpu.make_async_copy(tab.at[pl.ds(0, _BPW // 4)],
                              stag.at[pl.ds(0, _BPW // 4)], sems[q]).wait()
      pltpu.sync_copy(stag, out.at[pl.ds(base, _BPW)])

  return k(user_ids, device_ids, user_table, device_table)


_TB = 2048


def _mlp_body(u_ref, d_ref, bid_ref, bt_ref, w1u_ref, w1d_ref, w1b_ref,
              b1_ref, w2_ref, b2_ref, w3_ref, b3_ref, o_ref):
  iota = lax.broadcasted_iota(jnp.int32, (_TB, 1024), 1)
  onehot = (bid_ref[...].reshape(_TB, 1) == iota).astype(jnp.float32)
  b = jnp.dot(onehot, bt_ref[...], preferred_element_type=jnp.float32)
  h = jnp.dot(u_ref[...], w1u_ref[...], preferred_element_type=jnp.float32)
  h = h + jnp.dot(d_ref[...], w1d_ref[...], preferred_element_type=jnp.float32)
  h = h + jnp.dot(b, w1b_ref[...], preferred_element_type=jnp.float32)
  h = jnp.maximum(h + b1_ref[...], 0.0)
  h2 = jnp.dot(h, w2_ref[...], preferred_element_type=jnp.float32)
  h2 = jnp.maximum(h2 + b2_ref[...], 0.0)
  o_ref[...] = jnp.sum(h2 * w3_ref[...], axis=1) + b3_ref[0, 0]


def _mlp(u, d, brand_ids, brand_table, W1, b1, W2, b2, W3, b3):
  w1u, w1d, w1b = W1[:EMB], W1[EMB:2 * EMB], W1[2 * EMB:]
  bt_pad = jnp.zeros((1024, EMB), jnp.float32).at[:N_BRAND].set(brand_table)
  grid = (BATCH // _TB,)
  full = lambda shape: pl.BlockSpec(shape, lambda i: (0, 0))
  tile = pl.BlockSpec((_TB, EMB), lambda i: (i, 0))
  return pl.pallas_call(
      _mlp_body,
      grid=grid,
      in_specs=[
          tile, tile,
          pl.BlockSpec((_TB,), lambda i: (i,)),
          full((1024, EMB)),
          full((EMB, 128)), full((EMB, 128)), full((EMB, 128)),
          full((1, 128)),
          full((128, 64)), full((1, 64)),
          full((1, 64)), full((1, 1)),
      ],
      out_specs=pl.BlockSpec((_TB,), lambda i: (i,)),
      out_shape=jax.ShapeDtypeStruct((BATCH,), jnp.float32),
  )(u, d, brand_ids, bt_pad, w1u, w1d, w1b, b1.reshape(1, 128), W2,
    b2.reshape(1, 64), W3.reshape(1, EMB), b3.reshape(1, 1))


def kernel(user_ids, device_ids, brand_ids, user_table, device_table,
           brand_table, W1, b1, W2, b2, W3, b3):
  u, d = _gather2(user_ids.astype(jnp.int32), device_ids.astype(jnp.int32),
                  user_table, device_table)
  return _mlp(u, d, brand_ids.astype(jnp.int32), brand_table,
              W1, b1, W2, b2, W3, b3)
